# Initial kernel scaffold; baseline (speedup 1.0000x reference)
import jax
import jax.numpy as jnp
from jax import lax
from jax.experimental import pallas as pl
from jax.experimental.pallas import tpu as pltpu
from jax.experimental.pallas import tpu_sc as plsc

BATCH = 1024
MAXLEN = 200
EMBED = 64
HALF = 100

NUM_CORES = 2
NUM_SUBCORES = 16
NUM_WORKERS = NUM_CORES * NUM_SUBCORES
SEQS_PER_WORKER = BATCH // NUM_WORKERS
LANES = 16


def _embed_kernel(x_hbm, tok_hbm, pos_hbm, out_hbm, idx_v, pos_v, rows_v, sem):
    wid = lax.axis_index("s") * NUM_CORES + lax.axis_index("c")
    base = wid * SEQS_PER_WORKER

    pltpu.sync_copy(x_hbm.at[pl.ds(base, SEQS_PER_WORKER)], idx_v)
    pltpu.sync_copy(pos_hbm, pos_v)

    @pl.loop(0, SEQS_PER_WORKER)
    def _(s):
        pltpu.async_copy(
            tok_hbm.at[idx_v.at[s, 0]], rows_v.at[pl.ds(0, HALF)], sem
        ).wait()
        pltpu.async_copy(
            tok_hbm.at[idx_v.at[s, 1]], rows_v.at[pl.ds(HALF, HALF)], sem
        ).wait()

        @pl.loop(0, MAXLEN)
        def _(r):
            for j in range(EMBED // LANES):
                c = pl.ds(j * LANES, LANES)
                rows_v[r, c] = rows_v[r, c] + pos_v[r, c]

        pltpu.sync_copy(rows_v, out_hbm.at[base + s])


@jax.jit
def kernel(x, token_table, pos_table):
    x3 = x.reshape(BATCH, 2, HALF).astype(jnp.int32)
    mesh = plsc.VectorSubcoreMesh(core_axis_name="c", subcore_axis_name="s")
    k = pl.kernel(
        _embed_kernel,
        out_type=jax.ShapeDtypeStruct((BATCH, MAXLEN, EMBED), jnp.float32),
        mesh=mesh,
        scratch_types=[
            pltpu.VMEM((SEQS_PER_WORKER, 2, HALF), jnp.int32),
            pltpu.VMEM((MAXLEN, EMBED), jnp.float32),
            pltpu.VMEM((MAXLEN, EMBED), jnp.float32),
            pltpu.SemaphoreType.DMA,
        ],
    )
    return k(x3, token_table, pos_table)


# speedup vs baseline: 2.5612x; 2.5612x over previous
import jax
import jax.numpy as jnp
from jax import lax
from jax.experimental import pallas as pl
from jax.experimental.pallas import tpu as pltpu
from jax.experimental.pallas import tpu_sc as plsc

BATCH = 1024
MAXLEN = 200
EMBED = 64
HALF = 100

NUM_CORES = 2
NUM_SUBCORES = 16
NUM_WORKERS = NUM_CORES * NUM_SUBCORES
SEQS_PER_WORKER = BATCH // NUM_WORKERS
LANES = 16


def _embed_kernel(x_hbm, tok_hbm, pos_hbm, out_hbm, idx_v, pos_v, rows_v, sem):
    wid = lax.axis_index("s") * NUM_CORES + lax.axis_index("c")
    base = wid * SEQS_PER_WORKER

    pltpu.sync_copy(x_hbm.at[pl.ds(base, SEQS_PER_WORKER)], idx_v)
    pltpu.sync_copy(pos_hbm, pos_v)

    @pl.loop(0, SEQS_PER_WORKER)
    def _(s):
        pltpu.async_copy(
            tok_hbm.at[idx_v.at[s, 0]], rows_v.at[pl.ds(0, HALF)], sem
        ).wait()
        pltpu.async_copy(
            tok_hbm.at[idx_v.at[s, 1]], rows_v.at[pl.ds(HALF, HALF)], sem
        ).wait()

        @pl.loop(0, MAXLEN)
        def _(r):
            for j in range(EMBED // LANES):
                c = pl.ds(j * LANES, LANES)
                rows_v[r, c] = rows_v[r, c] + pos_v[r, c]

        pltpu.sync_copy(rows_v, out_hbm.at[base + s])


@jax.jit
def kernel(x, token_table, pos_table):
    x3 = x.reshape(BATCH, 2, HALF).astype(jnp.int32)
    mesh = plsc.VectorSubcoreMesh(core_axis_name="c", subcore_axis_name="s")
    k = pl.kernel(
        _embed_kernel,
        out_type=jax.ShapeDtypeStruct((BATCH, MAXLEN, EMBED), jnp.float32),
        mesh=mesh,
        scratch_types=[
            pltpu.VMEM((SEQS_PER_WORKER, 2, HALF), jnp.int32),
            pltpu.VMEM((MAXLEN, EMBED), jnp.float32),
            pltpu.VMEM((MAXLEN, EMBED), jnp.float32),
            pltpu.SemaphoreType.DMA,
        ],
        compiler_params=pltpu.CompilerParams(use_tc_tiling_on_sc=False),
    )
    return k(x3, token_table, pos_table)


# speedup vs baseline: 3.1915x; 1.2461x over previous
import jax
import jax.numpy as jnp
from jax import lax
from jax.experimental import pallas as pl
from jax.experimental.pallas import tpu as pltpu
from jax.experimental.pallas import tpu_sc as plsc

BATCH = 1024
MAXLEN = 200
EMBED = 64
HALF = 100

NUM_CORES = 2
NUM_SUBCORES = 16
NUM_WORKERS = NUM_CORES * NUM_SUBCORES
SEQS_PER_WORKER = BATCH // NUM_WORKERS
LANES = 16
NBUF = 2
NROUNDS = SEQS_PER_WORKER // NBUF


def _embed_kernel(x_hbm, tok_hbm, pos_hbm, out_hbm, idx_v, pos_v,
                  gbufs, wbufs, gsems, osems):
    wid = lax.axis_index("s") * NUM_CORES + lax.axis_index("c")
    base = wid * SEQS_PER_WORKER

    pltpu.sync_copy(x_hbm.at[pl.ds(base, SEQS_PER_WORKER)], idx_v)
    pltpu.sync_copy(pos_hbm, pos_v)

    def start_gather(s, b):
        pltpu.make_async_copy(
            tok_hbm.at[idx_v.at[s, 0]], gbufs[b].at[pl.ds(0, HALF)], gsems[b]
        ).start()
        pltpu.make_async_copy(
            tok_hbm.at[idx_v.at[s, 1]], gbufs[b].at[pl.ds(HALF, HALF)], gsems[b]
        ).start()

    def wait_gather(s, b):
        pltpu.make_async_copy(
            tok_hbm.at[idx_v.at[s, 0]], gbufs[b].at[pl.ds(0, HALF)], gsems[b]
        ).wait()
        pltpu.make_async_copy(
            tok_hbm.at[idx_v.at[s, 1]], gbufs[b].at[pl.ds(HALF, HALF)], gsems[b]
        ).wait()

    def add_pos(b):
        @pl.loop(0, MAXLEN)
        def _(r):
            for j in range(EMBED // LANES):
                c = pl.ds(j * LANES, LANES)
                wbufs[b][r, c] = gbufs[b][r, c] + pos_v[r, c]

    def start_write(s, b):
        pltpu.make_async_copy(wbufs[b], out_hbm.at[base + s], osems[b]).start()

    def wait_write(s, b):
        pltpu.make_async_copy(wbufs[b], out_hbm.at[base + s], osems[b]).wait()

    for b in range(NBUF):
        start_gather(b, b)

    for b in range(NBUF):
        wait_gather(b, b)
        add_pos(b)
        start_gather(NBUF + b, b)
        start_write(b, b)

    @pl.loop(1, NROUNDS - 1)
    def _(g):
        for b in range(NBUF):
            s = g * NBUF + b
            wait_gather(s, b)
            wait_write(s - NBUF, b)
            add_pos(b)
            start_gather(s + NBUF, b)
            start_write(s, b)

    for b in range(NBUF):
        s = (NROUNDS - 1) * NBUF + b
        wait_gather(s, b)
        wait_write(s - NBUF, b)
        add_pos(b)
        start_write(s, b)
    for b in range(NBUF):
        s = (NROUNDS - 1) * NBUF + b
        wait_write(s, b)


def _wrapped(x3, token_table, pos_table):
    mesh = plsc.VectorSubcoreMesh(core_axis_name="c", subcore_axis_name="s")
    vmem_rows = lambda: pltpu.VMEM((MAXLEN, EMBED), jnp.float32)

    def body(x_hbm, tok_hbm, pos_hbm, out_hbm, idx_v, pos_v,
             g0, g1, w0, w1, gs0, gs1, os0, os1):
        _embed_kernel(x_hbm, tok_hbm, pos_hbm, out_hbm, idx_v, pos_v,
                      (g0, g1), (w0, w1), (gs0, gs1), (os0, os1))

    k = pl.kernel(
        body,
        out_type=jax.ShapeDtypeStruct((BATCH, MAXLEN, EMBED), jnp.float32),
        mesh=mesh,
        scratch_types=[
            pltpu.VMEM((SEQS_PER_WORKER, 2, HALF), jnp.int32),
            vmem_rows(), vmem_rows(), vmem_rows(), vmem_rows(), vmem_rows(),
            pltpu.SemaphoreType.DMA,
            pltpu.SemaphoreType.DMA,
            pltpu.SemaphoreType.DMA,
            pltpu.SemaphoreType.DMA,
        ],
        compiler_params=pltpu.CompilerParams(use_tc_tiling_on_sc=False),
    )
    return k(x3, token_table, pos_table)


@jax.jit
def kernel(x, token_table, pos_table):
    x3 = x.reshape(BATCH, 2, HALF).astype(jnp.int32)
    return _wrapped(x3, token_table, pos_table)


# speedup vs baseline: 3.1919x; 1.0001x over previous
import jax
import jax.numpy as jnp
from jax import lax
from jax.experimental import pallas as pl
from jax.experimental.pallas import tpu as pltpu
from jax.experimental.pallas import tpu_sc as plsc

BATCH = 1024
MAXLEN = 200
EMBED = 64
HALF = 100

NUM_CORES = 2
NUM_SUBCORES = 16
NUM_WORKERS = NUM_CORES * NUM_SUBCORES
SEQS_PER_WORKER = BATCH // NUM_WORKERS
LANES = 16
NBUF = 2
NROUNDS = SEQS_PER_WORKER // NBUF


def _embed_kernel(x_hbm, tok_hbm, pos_hbm, out_hbm, idx_v, pos_v,
                  gbufs, wbufs, gsems, osems):
    wid = lax.axis_index("s") * NUM_CORES + lax.axis_index("c")
    base = wid * SEQS_PER_WORKER

    pltpu.sync_copy(x_hbm.at[pl.ds(base, SEQS_PER_WORKER)], idx_v)
    pltpu.sync_copy(pos_hbm, pos_v)

    def start_gather(s, b):
        pltpu.make_async_copy(
            tok_hbm.at[idx_v.at[s, 0]], gbufs[b].at[pl.ds(0, HALF)], gsems[b]
        ).start()
        pltpu.make_async_copy(
            tok_hbm.at[idx_v.at[s, 1]], gbufs[b].at[pl.ds(HALF, HALF)], gsems[b]
        ).start()

    def wait_gather(s, b):
        pltpu.make_async_copy(
            tok_hbm.at[idx_v.at[s, 0]], gbufs[b].at[pl.ds(0, HALF)], gsems[b]
        ).wait()
        pltpu.make_async_copy(
            tok_hbm.at[idx_v.at[s, 1]], gbufs[b].at[pl.ds(HALF, HALF)], gsems[b]
        ).wait()

    def add_pos(b):
        @pl.loop(0, HALF)
        def _(h):
            r = 2 * h
            for j in range(EMBED // LANES):
                c = pl.ds(j * LANES, LANES)
                cl = pl.ds(j * LANES + EMBED, LANES)
                wbufs[b][h, c] = gbufs[b][r, c] + pos_v[r, c]
                wbufs[b][h, cl] = gbufs[b][r + 1, c] + pos_v[r + 1, c]

    def start_write(s, b):
        pltpu.make_async_copy(
            wbufs[b], out_hbm.at[pl.ds((base + s) * HALF, HALF)], osems[b]
        ).start()

    def wait_write(s, b):
        pltpu.make_async_copy(
            wbufs[b], out_hbm.at[pl.ds((base + s) * HALF, HALF)], osems[b]
        ).wait()

    for b in range(NBUF):
        start_gather(b, b)

    for b in range(NBUF):
        wait_gather(b, b)
        add_pos(b)
        start_gather(NBUF + b, b)
        start_write(b, b)

    @pl.loop(1, NROUNDS - 1)
    def _(g):
        for b in range(NBUF):
            s = g * NBUF + b
            wait_gather(s, b)
            wait_write(s - NBUF, b)
            add_pos(b)
            start_gather(s + NBUF, b)
            start_write(s, b)

    for b in range(NBUF):
        s = (NROUNDS - 1) * NBUF + b
        wait_gather(s, b)
        wait_write(s - NBUF, b)
        add_pos(b)
        start_write(s, b)
    for b in range(NBUF):
        s = (NROUNDS - 1) * NBUF + b
        wait_write(s, b)


def _wrapped(x3, token_table, pos_table):
    mesh = plsc.VectorSubcoreMesh(core_axis_name="c", subcore_axis_name="s")
    vmem_rows = lambda: pltpu.VMEM((MAXLEN, EMBED), jnp.float32)

    def body(x_hbm, tok_hbm, pos_hbm, out_hbm, idx_v, pos_v,
             g0, g1, w0, w1, gs0, gs1, os0, os1):
        _embed_kernel(x_hbm, tok_hbm, pos_hbm, out_hbm, idx_v, pos_v,
                      (g0, g1), (w0, w1), (gs0, gs1), (os0, os1))

    k = pl.kernel(
        body,
        out_type=jax.ShapeDtypeStruct((BATCH * HALF, 2 * EMBED), jnp.float32),
        mesh=mesh,
        scratch_types=[
            pltpu.VMEM((SEQS_PER_WORKER, 2, HALF), jnp.int32),
            vmem_rows(), vmem_rows(), vmem_rows(),
            pltpu.VMEM((HALF, 2 * EMBED), jnp.float32),
            pltpu.VMEM((HALF, 2 * EMBED), jnp.float32),
            pltpu.SemaphoreType.DMA,
            pltpu.SemaphoreType.DMA,
            pltpu.SemaphoreType.DMA,
            pltpu.SemaphoreType.DMA,
        ],
        compiler_params=pltpu.CompilerParams(use_tc_tiling_on_sc=False),
    )
    return k(x3, token_table, pos_table)


@jax.jit
def kernel(x, token_table, pos_table):
    x3 = x.reshape(BATCH, 2, HALF).astype(jnp.int32)
    out = _wrapped(x3, token_table, pos_table)
    return out.reshape(BATCH, MAXLEN, EMBED)
